# baseline (speedup 1.0000x reference)
import functools

import jax
import jax.numpy as jnp
from jax import lax
from jax.experimental import pallas as pl
from jax.experimental.pallas import tpu as pltpu
from jax.experimental.pallas import tpu_sc as plsc

N = 10000
E = 320000
D_IN = 128
D_OUT = 64
B = 100
NPG = N // B

NC = 2
NS = 16
L = 16
NW = NC * NS
EPW = E // NW
CHUNK = 400
NCHUNK = EPW // CHUNK
WB_TILES = 10
RPT = N // WB_TILES
ZROWS = 200


def _tc_prep_body(feat_ref, anch_feat_ref, w_ref, b_ref, infeat_ref, anch_ref):
    prod = jnp.dot(feat_ref[...], w_ref[...], preferred_element_type=jnp.float32)
    row = lax.broadcasted_iota(jnp.int32, (N, 1), 0)
    infeat_ref[...] = jnp.where(row % NPG == 0, 0.0, prod)
    a = jnp.dot(anch_feat_ref[...], w_ref[...], preferred_element_type=jnp.float32)
    a = jnp.maximum(a + b_ref[...], 0.0)
    nrm = jnp.sqrt(jnp.sum(a * a, axis=1, keepdims=True))
    anch_ref[...] = a / jnp.maximum(nrm, 1e-12)


_tc_prep = pl.pallas_call(
    _tc_prep_body,
    out_shape=[
        jax.ShapeDtypeStruct((N, D_OUT), jnp.float32),
        jax.ShapeDtypeStruct((B, D_OUT), jnp.float32),
    ],
)


def _tc_final_body(part_ref, b_ref, hn_ref, pooled_ref):
    h = jnp.maximum(part_ref[0] + part_ref[1] + b_ref[...], 0.0)
    nrm = jnp.sqrt(jnp.sum(h * h, axis=1, keepdims=True))
    hn_ref[...] = h / jnp.maximum(nrm, 1e-12)
    p = jnp.sum(h.reshape(B, NPG, D_OUT), axis=1) * (1.0 / NPG)
    pn = jnp.sqrt(jnp.sum(p * p, axis=1, keepdims=True))
    pooled_ref[...] = p / jnp.maximum(pn, 1e-12)


_tc_final = pl.pallas_call(
    _tc_final_body,
    out_shape=[
        jax.ShapeDtypeStruct((N, D_OUT), jnp.float32),
        jax.ShapeDtypeStruct((B, D_OUT), jnp.float32),
    ],
)


_sc_mesh = plsc.VectorSubcoreMesh(core_axis_name="c", subcore_axis_name="s")


@functools.partial(
    pl.kernel,
    out_type=jax.ShapeDtypeStruct((NC * N, D_OUT), jnp.float32),
    mesh=_sc_mesh,
    compiler_params=pltpu.CompilerParams(use_tc_tiling_on_sc=False),
    scratch_types=[
        pltpu.VMEM((CHUNK,), jnp.int32),
        pltpu.VMEM((CHUNK,), jnp.int32),
        pltpu.VMEM((CHUNK,), jnp.int32),
        pltpu.VMEM((CHUNK,), jnp.int32),
        pltpu.VMEM((CHUNK,), jnp.float32),
        pltpu.VMEM((CHUNK,), jnp.float32),
        pltpu.VMEM((CHUNK, D_OUT), jnp.float32),
        pltpu.VMEM((CHUNK, D_OUT), jnp.float32),
        pltpu.VMEM((ZROWS, D_OUT), jnp.float32),
        pltpu.VMEM_SHARED((N, D_OUT), jnp.float32),
        pltpu.SemaphoreType.DMA,
        pltpu.SemaphoreType.DMA,
        pltpu.SemaphoreType.DMA,
        pltpu.SemaphoreType.DMA,
        pltpu.SemaphoreType.DMA,
        pltpu.SemaphoreType.DMA,
        pltpu.SemaphoreType.DMA,
        pltpu.SemaphoreType.DMA,
    ],
)
def _sc_edges(infeat_hbm, src2_hbm, dst2_hbm, wt2_hbm, out_hbm,
              src0, src1, dst0, dst1, wt0, wt1, rows0, rows1, zero_v, h_sh,
              ss0, ss1, sd0, sd1, sw0, sw1, sg0, sg1):
    src = (src0, src1)
    dst = (dst0, dst1)
    wt = (wt0, wt1)
    rows = (rows0, rows1)
    ss = (ss0, ss1)
    sd = (sd0, sd1)
    sw = (sw0, sw1)
    sg = (sg0, sg1)

    cid = lax.axis_index("c")
    sid = lax.axis_index("s")
    rbase = (cid * NS + sid) * NCHUNK

    z16 = jnp.zeros((L,), jnp.float32)
    r0 = sid * RPT

    @pl.when(sid < WB_TILES)
    def _zero():
        def zrow(i, carry):
            for j in range(D_OUT // L):
                zero_v[i, pl.ds(j * L, L)] = z16
            return carry

        lax.fori_loop(0, ZROWS, zrow, 0)
        for t in range(RPT // ZROWS):
            pltpu.sync_copy(zero_v, h_sh.at[pl.ds(r0 + t * ZROWS, ZROWS)])

    plsc.subcore_barrier()

    def fetch(row, b):
        pltpu.async_copy(src2_hbm.at[row], src[b], ss[b])
        pltpu.async_copy(dst2_hbm.at[row], dst[b], sd[b])
        pltpu.async_copy(wt2_hbm.at[row], wt[b], sw[b])
        pltpu.make_async_copy(src2_hbm.at[row], src[b], ss[b]).wait()
        pltpu.async_copy(infeat_hbm.at[src[b]], rows[b], sg[b])

    def process(row, b):
        pltpu.make_async_copy(infeat_hbm.at[src[b]], rows[b], sg[b]).wait()
        pltpu.make_async_copy(wt2_hbm.at[row], wt[b], sw[b]).wait()

        def group(g, carry):
            w16 = wt[b][pl.ds(g * L, L)]
            for t in range(L):
                wb = jnp.full((L,), w16[t])
                e = g * L + t
                for j in range(D_OUT // L):
                    rows[b][e, pl.ds(j * L, L)] = (
                        rows[b][e, pl.ds(j * L, L)] * wb)
            return carry

        lax.fori_loop(0, CHUNK // L, group, 0)
        pltpu.make_async_copy(dst2_hbm.at[row], dst[b], sd[b]).wait()
        pltpu.sync_copy(rows[b], h_sh.at[dst[b]], add=True)

    fetch(rbase, 0)

    def pair_body(j, carry):
        k = 2 * j
        fetch(rbase + k + 1, 1)
        process(rbase + k, 0)
        fetch(rbase + k + 2, 0)
        process(rbase + k + 1, 1)
        return carry

    lax.fori_loop(0, (NCHUNK - 1) // 2, pair_body, 0)
    process(rbase + NCHUNK - 1, 0)

    plsc.subcore_barrier()

    @pl.when(sid < WB_TILES)
    def _writeback():
        pltpu.sync_copy(h_sh.at[pl.ds(r0, RPT)],
                        out_hbm.at[pl.ds(cid * N + r0, RPT)])


def kernel(feat, edge_index, edge_weight, node_graph_ids, W, b):
    del node_graph_ids
    anchor_feat = feat[::NPG]
    b2 = b.reshape(1, D_OUT)
    in_feat, anchor_norm = _tc_prep(feat, anchor_feat, W, b2)
    src2 = edge_index[0].reshape(E // CHUNK, CHUNK)
    dst2 = edge_index[1].reshape(E // CHUNK, CHUNK)
    wt2 = edge_weight.reshape(E // CHUNK, CHUNK)
    parts = _sc_edges(in_feat, src2, dst2, wt2)
    parts = parts.reshape(NC, N, D_OUT)
    h_norm, pooled_norm = _tc_final(parts, b2)
    return (h_norm, pooled_norm, anchor_norm)


# speedup vs baseline: 1.4477x; 1.4477x over previous
import functools

import jax
import jax.numpy as jnp
from jax import lax
from jax.experimental import pallas as pl
from jax.experimental.pallas import tpu as pltpu
from jax.experimental.pallas import tpu_sc as plsc

N = 10000
E = 320000
D_IN = 128
D_OUT = 64
B = 100
NPG = N // B

NC = 2
NS = 16
L = 16
NW = NC * NS
EPW = E // NW
CHUNK = 80
NCHUNK = EPW // CHUNK
WB_TILES = 10
RPT = N // WB_TILES
ZROWS = 200


def _tc_prep_body(feat_ref, anch_feat_ref, w_ref, b_ref, infeat_ref, anch_ref):
    prod = jnp.dot(feat_ref[...], w_ref[...], preferred_element_type=jnp.float32)
    row = lax.broadcasted_iota(jnp.int32, (N, 1), 0)
    infeat_ref[...] = jnp.where(row % NPG == 0, 0.0, prod)
    a = jnp.dot(anch_feat_ref[...], w_ref[...], preferred_element_type=jnp.float32)
    a = jnp.maximum(a + b_ref[...], 0.0)
    nrm = jnp.sqrt(jnp.sum(a * a, axis=1, keepdims=True))
    anch_ref[...] = a / jnp.maximum(nrm, 1e-12)


_tc_prep = pl.pallas_call(
    _tc_prep_body,
    out_shape=[
        jax.ShapeDtypeStruct((N, D_OUT), jnp.float32),
        jax.ShapeDtypeStruct((B, D_OUT), jnp.float32),
    ],
)


def _tc_final_body(part_ref, b_ref, hn_ref, pooled_ref):
    h = jnp.maximum(part_ref[0] + part_ref[1] + b_ref[...], 0.0)
    nrm = jnp.sqrt(jnp.sum(h * h, axis=1, keepdims=True))
    hn_ref[...] = h / jnp.maximum(nrm, 1e-12)
    p = jnp.sum(h.reshape(B, NPG, D_OUT), axis=1) * (1.0 / NPG)
    pn = jnp.sqrt(jnp.sum(p * p, axis=1, keepdims=True))
    pooled_ref[...] = p / jnp.maximum(pn, 1e-12)


_tc_final = pl.pallas_call(
    _tc_final_body,
    out_shape=[
        jax.ShapeDtypeStruct((N, D_OUT), jnp.float32),
        jax.ShapeDtypeStruct((B, D_OUT), jnp.float32),
    ],
)


_sc_mesh = plsc.VectorSubcoreMesh(core_axis_name="c", subcore_axis_name="s")


@functools.partial(
    pl.kernel,
    out_type=jax.ShapeDtypeStruct((NC * N, D_OUT), jnp.float32),
    mesh=_sc_mesh,
    compiler_params=pltpu.CompilerParams(use_tc_tiling_on_sc=False),
    scratch_types=[
        pltpu.VMEM((CHUNK,), jnp.int32),
        pltpu.VMEM((CHUNK,), jnp.int32),
        pltpu.VMEM((CHUNK,), jnp.int32),
        pltpu.VMEM((CHUNK,), jnp.int32),
        pltpu.VMEM((CHUNK,), jnp.float32),
        pltpu.VMEM((CHUNK,), jnp.float32),
        pltpu.VMEM((CHUNK, D_OUT), jnp.float32),
        pltpu.VMEM((CHUNK, D_OUT), jnp.float32),
        pltpu.VMEM((ZROWS, D_OUT), jnp.float32),
        pltpu.VMEM_SHARED((N, D_OUT), jnp.float32),
        pltpu.SemaphoreType.DMA,
        pltpu.SemaphoreType.DMA,
        pltpu.SemaphoreType.DMA,
        pltpu.SemaphoreType.DMA,
        pltpu.SemaphoreType.DMA,
        pltpu.SemaphoreType.DMA,
        pltpu.SemaphoreType.DMA,
        pltpu.SemaphoreType.DMA,
    ],
)
def _sc_edges(infeat_hbm, src2_hbm, dst2_hbm, wt2_hbm, out_hbm,
              src0, src1, dst0, dst1, wt0, wt1, rows0, rows1, zero_v, h_sh,
              ss0, ss1, sd0, sd1, sw0, sw1, sg0, sg1):
    src = (src0, src1)
    dst = (dst0, dst1)
    wt = (wt0, wt1)
    rows = (rows0, rows1)
    ss = (ss0, ss1)
    sd = (sd0, sd1)
    sw = (sw0, sw1)
    sg = (sg0, sg1)

    cid = lax.axis_index("c")
    sid = lax.axis_index("s")
    rbase = (cid * NS + sid) * NCHUNK

    z16 = jnp.zeros((L,), jnp.float32)
    r0 = sid * RPT

    @pl.when(sid < WB_TILES)
    def _zero():
        def zrow(i, carry):
            for j in range(D_OUT // L):
                zero_v[i, pl.ds(j * L, L)] = z16
            return carry

        lax.fori_loop(0, ZROWS, zrow, 0)
        for t in range(RPT // ZROWS):
            pltpu.sync_copy(zero_v, h_sh.at[pl.ds(r0 + t * ZROWS, ZROWS)])

    plsc.subcore_barrier()

    def fetch(row, b):
        pltpu.async_copy(src2_hbm.at[row], src[b], ss[b])
        pltpu.async_copy(dst2_hbm.at[row], dst[b], sd[b])
        pltpu.async_copy(wt2_hbm.at[row], wt[b], sw[b])
        pltpu.make_async_copy(src2_hbm.at[row], src[b], ss[b]).wait()
        pltpu.async_copy(infeat_hbm.at[src[b]], rows[b], sg[b])

    def process(row, b):
        pltpu.make_async_copy(infeat_hbm.at[src[b]], rows[b], sg[b]).wait()
        pltpu.make_async_copy(wt2_hbm.at[row], wt[b], sw[b]).wait()

        for g in range(CHUNK // L):
            w16 = wt[b][pl.ds(g * L, L)]
            for t in range(L):
                wb = jnp.full((L,), w16[t])
                e = g * L + t
                for j in range(D_OUT // L):
                    rows[b][e, pl.ds(j * L, L)] = (
                        rows[b][e, pl.ds(j * L, L)] * wb)

        pltpu.make_async_copy(dst2_hbm.at[row], dst[b], sd[b]).wait()
        pltpu.sync_copy(rows[b], h_sh.at[dst[b]], add=True)

    fetch(rbase, 0)

    def pair_body(j, carry):
        k = 2 * j
        fetch(rbase + k + 1, 1)
        process(rbase + k, 0)
        fetch(rbase + k + 2, 0)
        process(rbase + k + 1, 1)
        return carry

    lax.fori_loop(0, (NCHUNK - 1) // 2, pair_body, 0)
    process(rbase + NCHUNK - 1, 0)

    plsc.subcore_barrier()

    @pl.when(sid < WB_TILES)
    def _writeback():
        pltpu.sync_copy(h_sh.at[pl.ds(r0, RPT)],
                        out_hbm.at[pl.ds(cid * N + r0, RPT)])


def kernel(feat, edge_index, edge_weight, node_graph_ids, W, b):
    del node_graph_ids
    anchor_feat = feat[::NPG]
    b2 = b.reshape(1, D_OUT)
    in_feat, anchor_norm = _tc_prep(feat, anchor_feat, W, b2)
    src2 = edge_index[0].reshape(E // CHUNK, CHUNK)
    dst2 = edge_index[1].reshape(E // CHUNK, CHUNK)
    wt2 = edge_weight.reshape(E // CHUNK, CHUNK)
    parts = _sc_edges(in_feat, src2, dst2, wt2)
    parts = parts.reshape(NC, N, D_OUT)
    h_norm, pooled_norm = _tc_final(parts, b2)
    return (h_norm, pooled_norm, anchor_norm)


# speedup vs baseline: 1.5979x; 1.1037x over previous
import functools

import jax
import jax.numpy as jnp
from jax import lax
from jax.experimental import pallas as pl
from jax.experimental.pallas import tpu as pltpu
from jax.experimental.pallas import tpu_sc as plsc

N = 10000
E = 320000
D_IN = 128
D_OUT = 64
B = 100
NPG = N // B

NC = 2
NS = 16
L = 16
NW = NC * NS
EPW = E // NW
CHUNK = 80
NCHUNK = EPW // CHUNK
WB_TILES = 10
RPT = N // WB_TILES
ZROWS = 200


def _tc_prep_body(feat_ref, anch_feat_ref, w_ref, b_ref, infeat_ref, anch_ref):
    prod = jnp.dot(feat_ref[...], w_ref[...], preferred_element_type=jnp.float32)
    row = lax.broadcasted_iota(jnp.int32, (N, 1), 0)
    infeat_ref[...] = jnp.where(row % NPG == 0, 0.0, prod)
    a = jnp.dot(anch_feat_ref[...], w_ref[...], preferred_element_type=jnp.float32)
    a = jnp.maximum(a + b_ref[...], 0.0)
    nrm = jnp.sqrt(jnp.sum(a * a, axis=1, keepdims=True))
    anch_ref[...] = a / jnp.maximum(nrm, 1e-12)


_tc_prep = pl.pallas_call(
    _tc_prep_body,
    out_shape=[
        jax.ShapeDtypeStruct((N, D_OUT), jnp.float32),
        jax.ShapeDtypeStruct((B, D_OUT), jnp.float32),
    ],
)


def _tc_final_body(part_ref, b_ref, hn_ref, pooled_ref):
    h = jnp.maximum(part_ref[0] + part_ref[1] + b_ref[...], 0.0)
    nrm = jnp.sqrt(jnp.sum(h * h, axis=1, keepdims=True))
    hn_ref[...] = h / jnp.maximum(nrm, 1e-12)
    p = jnp.sum(h.reshape(B, NPG, D_OUT), axis=1) * (1.0 / NPG)
    pn = jnp.sqrt(jnp.sum(p * p, axis=1, keepdims=True))
    pooled_ref[...] = p / jnp.maximum(pn, 1e-12)


_tc_final = pl.pallas_call(
    _tc_final_body,
    out_shape=[
        jax.ShapeDtypeStruct((N, D_OUT), jnp.float32),
        jax.ShapeDtypeStruct((B, D_OUT), jnp.float32),
    ],
)


_sc_mesh = plsc.VectorSubcoreMesh(core_axis_name="c", subcore_axis_name="s")


@functools.partial(
    pl.kernel,
    out_type=jax.ShapeDtypeStruct((NC * N, D_OUT), jnp.float32),
    mesh=_sc_mesh,
    compiler_params=pltpu.CompilerParams(use_tc_tiling_on_sc=False),
    scratch_types=[
        pltpu.VMEM((CHUNK,), jnp.int32),
        pltpu.VMEM((CHUNK,), jnp.int32),
        pltpu.VMEM((CHUNK,), jnp.int32),
        pltpu.VMEM((CHUNK,), jnp.int32),
        pltpu.VMEM((CHUNK,), jnp.int32),
        pltpu.VMEM((CHUNK,), jnp.int32),
        pltpu.VMEM((CHUNK,), jnp.float32),
        pltpu.VMEM((CHUNK,), jnp.float32),
        pltpu.VMEM((CHUNK,), jnp.float32),
        pltpu.VMEM((CHUNK, D_OUT), jnp.float32),
        pltpu.VMEM((CHUNK, D_OUT), jnp.float32),
        pltpu.VMEM((CHUNK, D_OUT), jnp.float32),
        pltpu.VMEM((ZROWS, D_OUT), jnp.float32),
        pltpu.VMEM_SHARED((N, D_OUT), jnp.float32),
        pltpu.SemaphoreType.DMA,
        pltpu.SemaphoreType.DMA,
        pltpu.SemaphoreType.DMA,
        pltpu.SemaphoreType.DMA,
        pltpu.SemaphoreType.DMA,
        pltpu.SemaphoreType.DMA,
        pltpu.SemaphoreType.DMA,
        pltpu.SemaphoreType.DMA,
        pltpu.SemaphoreType.DMA,
        pltpu.SemaphoreType.DMA,
        pltpu.SemaphoreType.DMA,
        pltpu.SemaphoreType.DMA,
        pltpu.SemaphoreType.DMA,
        pltpu.SemaphoreType.DMA,
        pltpu.SemaphoreType.DMA,
    ],
)
def _sc_edges(infeat_hbm, src2_hbm, dst2_hbm, wt2_hbm, out_hbm,
              src0, src1, src2, dst0, dst1, dst2, wt0, wt1, wt2,
              rows0, rows1, rows2, zero_v, h_sh,
              ss0, ss1, ss2, sd0, sd1, sd2, sw0, sw1, sw2,
              sg0, sg1, sg2, sc0, sc1, sc2):
    src = (src0, src1, src2)
    dst = (dst0, dst1, dst2)
    wt = (wt0, wt1, wt2)
    rows = (rows0, rows1, rows2)
    ss = (ss0, ss1, ss2)
    sd = (sd0, sd1, sd2)
    sw = (sw0, sw1, sw2)
    sg = (sg0, sg1, sg2)
    sc = (sc0, sc1, sc2)

    cid = lax.axis_index("c")
    sid = lax.axis_index("s")
    rbase = (cid * NS + sid) * NCHUNK

    z16 = jnp.zeros((L,), jnp.float32)
    r0 = sid * RPT

    @pl.when(sid < WB_TILES)
    def _zero():
        def zrow(i, carry):
            for j in range(D_OUT // L):
                zero_v[i, pl.ds(j * L, L)] = z16
            return carry

        lax.fori_loop(0, ZROWS, zrow, 0)
        for t in range(RPT // ZROWS):
            pltpu.sync_copy(zero_v, h_sh.at[pl.ds(r0 + t * ZROWS, ZROWS)])

    plsc.subcore_barrier()

    def fetch(row, b):
        pltpu.async_copy(src2_hbm.at[row], src[b], ss[b])
        pltpu.async_copy(dst2_hbm.at[row], dst[b], sd[b])
        pltpu.async_copy(wt2_hbm.at[row], wt[b], sw[b])
        pltpu.make_async_copy(src2_hbm.at[row], src[b], ss[b]).wait()
        pltpu.async_copy(infeat_hbm.at[src[b]], rows[b], sg[b])

    def process(row, b):
        pltpu.make_async_copy(infeat_hbm.at[src[b]], rows[b], sg[b]).wait()
        pltpu.make_async_copy(wt2_hbm.at[row], wt[b], sw[b]).wait()

        for g in range(CHUNK // L):
            w16 = wt[b][pl.ds(g * L, L)]
            for t in range(L):
                wb = jnp.full((L,), w16[t])
                e = g * L + t
                for j in range(D_OUT // L):
                    rows[b][e, pl.ds(j * L, L)] = (
                        rows[b][e, pl.ds(j * L, L)] * wb)

        pltpu.make_async_copy(dst2_hbm.at[row], dst[b], sd[b]).wait()
        pltpu.async_copy(rows[b], h_sh.at[dst[b]], sc[b], add=True)

    def scat_wait(b):
        pltpu.make_async_copy(rows[b], h_sh.at[dst[b]], sc[b]).wait()

    fetch(rbase, 0)
    fetch(rbase + 1, 1)
    process(rbase, 0)
    fetch(rbase + 2, 2)

    def tri_body(j, carry):
        k3 = 3 * j + 1
        for t in range(3):
            k = k3 + t
            s = (1 + t) % 3
            process(rbase + k, s)

            @pl.when(k + 2 <= NCHUNK - 1)
            def _prefetch():
                nb = (s + 2) % 3
                scat_wait(nb)
                fetch(rbase + k + 2, nb)
        return carry

    lax.fori_loop(0, (NCHUNK - 2) // 3, tri_body, 0)
    process(rbase + NCHUNK - 1, (NCHUNK - 1) % 3)
    for s in range(3):
        scat_wait(s)

    plsc.subcore_barrier()

    @pl.when(sid < WB_TILES)
    def _writeback():
        pltpu.sync_copy(h_sh.at[pl.ds(r0, RPT)],
                        out_hbm.at[pl.ds(cid * N + r0, RPT)])


def kernel(feat, edge_index, edge_weight, node_graph_ids, W, b):
    del node_graph_ids
    anchor_feat = feat[::NPG]
    b2 = b.reshape(1, D_OUT)
    in_feat, anchor_norm = _tc_prep(feat, anchor_feat, W, b2)
    src2 = edge_index[0].reshape(E // CHUNK, CHUNK)
    dst2 = edge_index[1].reshape(E // CHUNK, CHUNK)
    wt2 = edge_weight.reshape(E // CHUNK, CHUNK)
    parts = _sc_edges(in_feat, src2, dst2, wt2)
    parts = parts.reshape(NC, N, D_OUT)
    h_norm, pooled_norm = _tc_final(parts, b2)
    return (h_norm, pooled_norm, anchor_norm)


# speedup vs baseline: 1.8878x; 1.1814x over previous
import functools

import jax
import jax.numpy as jnp
from jax import lax
from jax.experimental import pallas as pl
from jax.experimental.pallas import tpu as pltpu
from jax.experimental.pallas import tpu_sc as plsc

N = 10000
E = 320000
D_IN = 128
D_OUT = 64
B = 100
NPG = N // B

NC = 2
NS = 16
L = 16
NW = NC * NS
EPW = E // NW
CHUNK = 80
NCHUNK = EPW // CHUNK
WB_TILES = 10
RPT = N // WB_TILES
ZROWS = 200


def _tc_prep_body(feat_ref, anch_feat_ref, w_ref, b_ref, infeat_ref, anch_ref):
    prod = jnp.dot(feat_ref[...], w_ref[...], preferred_element_type=jnp.float32)
    row = lax.broadcasted_iota(jnp.int32, (N, 1), 0)
    infeat_ref[...] = jnp.where(row % NPG == 0, 0.0, prod)
    a = jnp.dot(anch_feat_ref[...], w_ref[...], preferred_element_type=jnp.float32)
    a = jnp.maximum(a + b_ref[...], 0.0)
    nrm = jnp.sqrt(jnp.sum(a * a, axis=1, keepdims=True))
    anch_ref[...] = a / jnp.maximum(nrm, 1e-12)


_tc_prep = pl.pallas_call(
    _tc_prep_body,
    out_shape=[
        jax.ShapeDtypeStruct((N, D_OUT), jnp.float32),
        jax.ShapeDtypeStruct((B, D_OUT), jnp.float32),
    ],
)


def _tc_final_body(part_ref, b_ref, hn_ref, pooled_ref):
    h = jnp.maximum(part_ref[0] + part_ref[1] + b_ref[...], 0.0)
    nrm = jnp.sqrt(jnp.sum(h * h, axis=1, keepdims=True))
    hn_ref[...] = h / jnp.maximum(nrm, 1e-12)
    p = jnp.sum(h.reshape(B, NPG, D_OUT), axis=1) * (1.0 / NPG)
    pn = jnp.sqrt(jnp.sum(p * p, axis=1, keepdims=True))
    pooled_ref[...] = p / jnp.maximum(pn, 1e-12)


_tc_final = pl.pallas_call(
    _tc_final_body,
    out_shape=[
        jax.ShapeDtypeStruct((N, D_OUT), jnp.float32),
        jax.ShapeDtypeStruct((B, D_OUT), jnp.float32),
    ],
)


_sc_mesh = plsc.VectorSubcoreMesh(core_axis_name="c", subcore_axis_name="s")


@functools.partial(
    pl.kernel,
    out_type=jax.ShapeDtypeStruct((NC * N, D_OUT), jnp.float32),
    mesh=_sc_mesh,
    compiler_params=pltpu.CompilerParams(use_tc_tiling_on_sc=False),
    scratch_types=[
        pltpu.VMEM((NCHUNK, CHUNK), jnp.int32),
        pltpu.VMEM((NCHUNK, CHUNK), jnp.int32),
        pltpu.VMEM((NCHUNK, CHUNK), jnp.float32),
        pltpu.VMEM((CHUNK, D_OUT), jnp.float32),
        pltpu.VMEM((CHUNK, D_OUT), jnp.float32),
        pltpu.VMEM((CHUNK, D_OUT), jnp.float32),
        pltpu.VMEM((ZROWS, D_OUT), jnp.float32),
        pltpu.VMEM_SHARED((N, D_OUT), jnp.float32),
        pltpu.SemaphoreType.DMA,
        pltpu.SemaphoreType.DMA,
        pltpu.SemaphoreType.DMA,
        pltpu.SemaphoreType.DMA,
        pltpu.SemaphoreType.DMA,
        pltpu.SemaphoreType.DMA,
        pltpu.SemaphoreType.DMA,
    ],
)
def _sc_edges(infeat_hbm, src2_hbm, dst2_hbm, wt2_hbm, out_hbm,
              src_all, dst_all, wt_all, rows0, rows1, rows2, zero_v, h_sh,
              st, sg0, sg1, sg2, sc0, sc1, sc2):
    rows = (rows0, rows1, rows2)
    sg = (sg0, sg1, sg2)
    sc = (sc0, sc1, sc2)

    cid = lax.axis_index("c")
    sid = lax.axis_index("s")
    rbase = (cid * NS + sid) * NCHUNK

    pltpu.async_copy(src2_hbm.at[pl.ds(rbase, NCHUNK)], src_all, st)
    pltpu.async_copy(dst2_hbm.at[pl.ds(rbase, NCHUNK)], dst_all, st)
    pltpu.async_copy(wt2_hbm.at[pl.ds(rbase, NCHUNK)], wt_all, st)

    z16 = jnp.zeros((L,), jnp.float32)
    r0 = sid * RPT

    @pl.when(sid < WB_TILES)
    def _zero():
        def zrow(i, carry):
            for j in range(D_OUT // L):
                zero_v[i, pl.ds(j * L, L)] = z16
            return carry

        lax.fori_loop(0, ZROWS, zrow, 0)
        for t in range(RPT // ZROWS):
            pltpu.sync_copy(zero_v, h_sh.at[pl.ds(r0 + t * ZROWS, ZROWS)])

    pltpu.make_async_copy(src2_hbm.at[pl.ds(rbase, NCHUNK)], src_all, st).wait()
    pltpu.make_async_copy(dst2_hbm.at[pl.ds(rbase, NCHUNK)], dst_all, st).wait()
    pltpu.make_async_copy(wt2_hbm.at[pl.ds(rbase, NCHUNK)], wt_all, st).wait()
    plsc.subcore_barrier()

    def gfetch(k, b):
        pltpu.async_copy(infeat_hbm.at[src_all.at[k]], rows[b], sg[b])

    def process(k, b):
        pltpu.make_async_copy(
            infeat_hbm.at[src_all.at[k]], rows[b], sg[b]).wait()

        for g in range(CHUNK // L):
            w16 = wt_all[k, pl.ds(g * L, L)]
            for t in range(L):
                wb = jnp.full((L,), w16[t])
                e = g * L + t
                for j in range(D_OUT // L):
                    rows[b][e, pl.ds(j * L, L)] = (
                        rows[b][e, pl.ds(j * L, L)] * wb)

        pltpu.async_copy(rows[b], h_sh.at[dst_all.at[k]], sc[b], add=True)

    def scat_wait(b):
        pltpu.make_async_copy(rows[b], h_sh.at[dst_all.at[0]], sc[b]).wait()

    gfetch(0, 0)
    gfetch(1, 1)
    process(0, 0)
    gfetch(2, 2)

    def tri_body(j, carry):
        k3 = 3 * j + 1
        for t in range(3):
            k = k3 + t
            s = (1 + t) % 3
            process(k, s)

            @pl.when(k + 2 <= NCHUNK - 1)
            def _prefetch():
                nb = (s + 2) % 3
                scat_wait(nb)
                gfetch(k + 2, nb)
        return carry

    lax.fori_loop(0, (NCHUNK - 2) // 3, tri_body, 0)
    process(NCHUNK - 1, (NCHUNK - 1) % 3)
    for s in range(3):
        scat_wait(s)

    plsc.subcore_barrier()

    @pl.when(sid < WB_TILES)
    def _writeback():
        pltpu.sync_copy(h_sh.at[pl.ds(r0, RPT)],
                        out_hbm.at[pl.ds(cid * N + r0, RPT)])


def kernel(feat, edge_index, edge_weight, node_graph_ids, W, b):
    del node_graph_ids
    anchor_feat = feat[::NPG]
    b2 = b.reshape(1, D_OUT)
    in_feat, anchor_norm = _tc_prep(feat, anchor_feat, W, b2)
    src2 = edge_index[0].reshape(E // CHUNK, CHUNK)
    dst2 = edge_index[1].reshape(E // CHUNK, CHUNK)
    wt2 = edge_weight.reshape(E // CHUNK, CHUNK)
    parts = _sc_edges(in_feat, src2, dst2, wt2)
    parts = parts.reshape(NC, N, D_OUT)
    h_norm, pooled_norm = _tc_final(parts, b2)
    return (h_norm, pooled_norm, anchor_norm)


# speedup vs baseline: 2.1083x; 1.1168x over previous
import functools

import jax
import jax.numpy as jnp
from jax import lax
from jax.experimental import pallas as pl
from jax.experimental.pallas import tpu as pltpu
from jax.experimental.pallas import tpu_sc as plsc

N = 10000
E = 320000
D_IN = 128
D_OUT = 64
B = 100
NPG = N // B

NC = 2
NS = 16
L = 16
NW = NC * NS
EPW = E // NW
CHUNK = 80
NCHUNK = EPW // CHUNK
WB_TILES = 10
RPT = N // WB_TILES
ZROWS = 200


def _tc_prep_body(feat_ref, anch_feat_ref, w_ref, b_ref, infeat_ref, anch_ref):
    prod = jnp.dot(feat_ref[...], w_ref[...], preferred_element_type=jnp.float32)
    row = lax.broadcasted_iota(jnp.int32, (N, 1), 0)
    infeat_ref[...] = jnp.where(row % NPG == 0, 0.0, prod)
    a = jnp.dot(anch_feat_ref[...], w_ref[...], preferred_element_type=jnp.float32)
    a = jnp.maximum(a + b_ref[...], 0.0)
    nrm = jnp.sqrt(jnp.sum(a * a, axis=1, keepdims=True))
    anch_ref[...] = a / jnp.maximum(nrm, 1e-12)


_tc_prep = pl.pallas_call(
    _tc_prep_body,
    out_shape=[
        jax.ShapeDtypeStruct((N, D_OUT), jnp.float32),
        jax.ShapeDtypeStruct((B, D_OUT), jnp.float32),
    ],
)


def _tc_final_body(part_ref, b_ref, hn_ref, pooled_ref):
    h = jnp.maximum(part_ref[0] + part_ref[1] + b_ref[...], 0.0)
    nrm = jnp.sqrt(jnp.sum(h * h, axis=1, keepdims=True))
    hn_ref[...] = h / jnp.maximum(nrm, 1e-12)
    p = jnp.sum(h.reshape(B, NPG, D_OUT), axis=1) * (1.0 / NPG)
    pn = jnp.sqrt(jnp.sum(p * p, axis=1, keepdims=True))
    pooled_ref[...] = p / jnp.maximum(pn, 1e-12)


_tc_final = pl.pallas_call(
    _tc_final_body,
    out_shape=[
        jax.ShapeDtypeStruct((N, D_OUT), jnp.float32),
        jax.ShapeDtypeStruct((B, D_OUT), jnp.float32),
    ],
)


_sc_mesh = plsc.VectorSubcoreMesh(core_axis_name="c", subcore_axis_name="s")


@functools.partial(
    pl.kernel,
    out_type=jax.ShapeDtypeStruct((NC * N, D_OUT), jnp.float32),
    mesh=_sc_mesh,
    compiler_params=pltpu.CompilerParams(use_tc_tiling_on_sc=False),
    scratch_types=[
        pltpu.VMEM((NCHUNK, CHUNK), jnp.int32),
        pltpu.VMEM((NCHUNK, CHUNK), jnp.int32),
        pltpu.VMEM((NCHUNK, CHUNK), jnp.float32),
        pltpu.VMEM((CHUNK, D_OUT), jnp.float32),
        pltpu.VMEM((CHUNK, D_OUT), jnp.float32),
        pltpu.VMEM((CHUNK, D_OUT), jnp.float32),
        pltpu.VMEM((CHUNK, D_OUT), jnp.float32),
        pltpu.VMEM((ZROWS, D_OUT), jnp.float32),
        pltpu.VMEM_SHARED((N, D_OUT), jnp.float32),
        pltpu.SemaphoreType.DMA,
        pltpu.SemaphoreType.DMA,
        pltpu.SemaphoreType.DMA,
        pltpu.SemaphoreType.DMA,
        pltpu.SemaphoreType.DMA,
        pltpu.SemaphoreType.DMA,
        pltpu.SemaphoreType.DMA,
        pltpu.SemaphoreType.DMA,
        pltpu.SemaphoreType.DMA,
    ],
)
def _sc_edges(infeat_hbm, src2_hbm, dst2_hbm, wt2_hbm, out_hbm,
              src_all, dst_all, wt_all, rows0, rows1, rows2, rows3, zero_v,
              h_sh, st, sg0, sg1, sg2, sg3, sc0, sc1, sc2, sc3):
    rows = (rows0, rows1, rows2, rows3)
    sg = (sg0, sg1, sg2, sg3)
    sc = (sc0, sc1, sc2, sc3)

    cid = lax.axis_index("c")
    sid = lax.axis_index("s")
    rbase = (cid * NS + sid) * NCHUNK

    pltpu.async_copy(src2_hbm.at[pl.ds(rbase, NCHUNK)], src_all, st)
    pltpu.async_copy(dst2_hbm.at[pl.ds(rbase, NCHUNK)], dst_all, st)
    pltpu.async_copy(wt2_hbm.at[pl.ds(rbase, NCHUNK)], wt_all, st)

    z16 = jnp.zeros((L,), jnp.float32)
    r0 = sid * RPT

    @pl.when(sid < WB_TILES)
    def _zero():
        def zrow(i, carry):
            for j in range(D_OUT // L):
                zero_v[i, pl.ds(j * L, L)] = z16
            return carry

        lax.fori_loop(0, ZROWS, zrow, 0)
        for t in range(RPT // ZROWS):
            pltpu.sync_copy(zero_v, h_sh.at[pl.ds(r0 + t * ZROWS, ZROWS)])

    pltpu.make_async_copy(src2_hbm.at[pl.ds(rbase, NCHUNK)], src_all, st).wait()
    pltpu.make_async_copy(dst2_hbm.at[pl.ds(rbase, NCHUNK)], dst_all, st).wait()
    pltpu.make_async_copy(wt2_hbm.at[pl.ds(rbase, NCHUNK)], wt_all, st).wait()
    plsc.subcore_barrier()

    def gfetch(k, b):
        pltpu.async_copy(infeat_hbm.at[src_all.at[k]], rows[b], sg[b])

    def process(k, b):
        pltpu.make_async_copy(
            infeat_hbm.at[src_all.at[k]], rows[b], sg[b]).wait()

        for g in range(CHUNK // L):
            w16 = wt_all[k, pl.ds(g * L, L)]
            for t in range(L):
                wb = jnp.full((L,), w16[t])
                e = g * L + t
                for j in range(D_OUT // L):
                    rows[b][e, pl.ds(j * L, L)] = (
                        rows[b][e, pl.ds(j * L, L)] * wb)

        pltpu.async_copy(rows[b], h_sh.at[dst_all.at[k]], sc[b], add=True)

    def scat_wait(b):
        pltpu.make_async_copy(rows[b], h_sh.at[dst_all.at[0]], sc[b]).wait()

    gfetch(0, 0)
    gfetch(1, 1)
    gfetch(2, 2)
    process(0, 0)
    gfetch(3, 3)

    def quad_body(j, carry):
        k4 = 4 * j + 1
        for t in range(4):
            k = k4 + t
            s = (1 + t) % 4
            process(k, s)

            @pl.when(k + 3 <= NCHUNK - 1)
            def _prefetch():
                nb = (s + 3) % 4
                scat_wait(nb)
                gfetch(k + 3, nb)
        return carry

    lax.fori_loop(0, (NCHUNK - 1) // 4, quad_body, 0)
    for s in range(4):
        scat_wait(s)

    plsc.subcore_barrier()

    @pl.when(sid < WB_TILES)
    def _writeback():
        pltpu.sync_copy(h_sh.at[pl.ds(r0, RPT)],
                        out_hbm.at[pl.ds(cid * N + r0, RPT)])


def kernel(feat, edge_index, edge_weight, node_graph_ids, W, b):
    del node_graph_ids
    anchor_feat = feat[::NPG]
    b2 = b.reshape(1, D_OUT)
    in_feat, anchor_norm = _tc_prep(feat, anchor_feat, W, b2)
    src2 = edge_index[0].reshape(E // CHUNK, CHUNK)
    dst2 = edge_index[1].reshape(E // CHUNK, CHUNK)
    wt2 = edge_weight.reshape(E // CHUNK, CHUNK)
    parts = _sc_edges(in_feat, src2, dst2, wt2)
    parts = parts.reshape(NC, N, D_OUT)
    h_norm, pooled_norm = _tc_final(parts, b2)
    return (h_norm, pooled_norm, anchor_norm)


# speedup vs baseline: 2.1481x; 1.0189x over previous
import functools

import jax
import jax.numpy as jnp
from jax import lax
from jax.experimental import pallas as pl
from jax.experimental.pallas import tpu as pltpu
from jax.experimental.pallas import tpu_sc as plsc

N = 10000
E = 320000
D_IN = 128
D_OUT = 64
B = 100
NPG = N // B

NC = 2
NS = 16
L = 16
NW = NC * NS
EPW = E // NW
CHUNK = 80
NCHUNK = EPW // CHUNK
WB_TILES = 10
RPT = N // WB_TILES
ZROWS = 200


def _tc_prep_body(feat_ref, w_ref, b_ref, infeat_ref, anch_ref):
    prod = jnp.dot(feat_ref[...], w_ref[...], preferred_element_type=jnp.float32)
    row = lax.broadcasted_iota(jnp.int32, (N, 1), 0)
    infeat_ref[...] = jnp.where(row % NPG == 0, 0.0, prod)
    a = prod.reshape(B, NPG, D_OUT)[:, 0, :]
    a = jnp.maximum(a + b_ref[...], 0.0)
    nrm = jnp.sqrt(jnp.sum(a * a, axis=1, keepdims=True))
    anch_ref[...] = a / jnp.maximum(nrm, 1e-12)


_tc_prep = pl.pallas_call(
    _tc_prep_body,
    out_shape=[
        jax.ShapeDtypeStruct((N, D_OUT), jnp.float32),
        jax.ShapeDtypeStruct((B, D_OUT), jnp.float32),
    ],
)


def _tc_final_body(part_ref, b_ref, hn_ref, pooled_ref):
    h = jnp.maximum(part_ref[0] + part_ref[1] + b_ref[...], 0.0)
    nrm = jnp.sqrt(jnp.sum(h * h, axis=1, keepdims=True))
    hn_ref[...] = h / jnp.maximum(nrm, 1e-12)
    p = jnp.sum(h.reshape(B, NPG, D_OUT), axis=1) * (1.0 / NPG)
    pn = jnp.sqrt(jnp.sum(p * p, axis=1, keepdims=True))
    pooled_ref[...] = p / jnp.maximum(pn, 1e-12)


_tc_final = pl.pallas_call(
    _tc_final_body,
    out_shape=[
        jax.ShapeDtypeStruct((N, D_OUT), jnp.float32),
        jax.ShapeDtypeStruct((B, D_OUT), jnp.float32),
    ],
)


_sc_mesh = plsc.VectorSubcoreMesh(core_axis_name="c", subcore_axis_name="s")


@functools.partial(
    pl.kernel,
    out_type=jax.ShapeDtypeStruct((NC * N, D_OUT), jnp.float32),
    mesh=_sc_mesh,
    compiler_params=pltpu.CompilerParams(use_tc_tiling_on_sc=False),
    scratch_types=[
        pltpu.VMEM((NCHUNK, CHUNK), jnp.int32),
        pltpu.VMEM((NCHUNK, CHUNK), jnp.int32),
        pltpu.VMEM((NCHUNK, CHUNK), jnp.float32),
        pltpu.VMEM((CHUNK, D_OUT), jnp.float32),
        pltpu.VMEM((CHUNK, D_OUT), jnp.float32),
        pltpu.VMEM((CHUNK, D_OUT), jnp.float32),
        pltpu.VMEM((CHUNK, D_OUT), jnp.float32),
        pltpu.VMEM((ZROWS, D_OUT), jnp.float32),
        pltpu.VMEM_SHARED((N, D_OUT), jnp.float32),
        pltpu.SemaphoreType.DMA,
        pltpu.SemaphoreType.DMA,
        pltpu.SemaphoreType.DMA,
        pltpu.SemaphoreType.DMA,
        pltpu.SemaphoreType.DMA,
        pltpu.SemaphoreType.DMA,
        pltpu.SemaphoreType.DMA,
        pltpu.SemaphoreType.DMA,
        pltpu.SemaphoreType.DMA,
    ],
)
def _sc_edges(infeat_hbm, src2_hbm, dst2_hbm, wt2_hbm, out_hbm,
              src_all, dst_all, wt_all, rows0, rows1, rows2, rows3, zero_v,
              h_sh, st, sg0, sg1, sg2, sg3, sc0, sc1, sc2, sc3):
    rows = (rows0, rows1, rows2, rows3)
    sg = (sg0, sg1, sg2, sg3)
    sc = (sc0, sc1, sc2, sc3)

    cid = lax.axis_index("c")
    sid = lax.axis_index("s")
    rbase = (cid * NS + sid) * NCHUNK

    pltpu.async_copy(src2_hbm.at[pl.ds(rbase, NCHUNK)], src_all, st)
    pltpu.async_copy(dst2_hbm.at[pl.ds(rbase, NCHUNK)], dst_all, st)
    pltpu.async_copy(wt2_hbm.at[pl.ds(rbase, NCHUNK)], wt_all, st)

    z16 = jnp.zeros((L,), jnp.float32)
    r0 = sid * RPT

    @pl.when(sid < WB_TILES)
    def _zero():
        def zrow(i, carry):
            for j in range(D_OUT // L):
                zero_v[i, pl.ds(j * L, L)] = z16
            return carry

        lax.fori_loop(0, ZROWS, zrow, 0)
        for t in range(RPT // ZROWS):
            pltpu.sync_copy(zero_v, h_sh.at[pl.ds(r0 + t * ZROWS, ZROWS)])

    pltpu.make_async_copy(src2_hbm.at[pl.ds(rbase, NCHUNK)], src_all, st).wait()
    pltpu.make_async_copy(dst2_hbm.at[pl.ds(rbase, NCHUNK)], dst_all, st).wait()
    pltpu.make_async_copy(wt2_hbm.at[pl.ds(rbase, NCHUNK)], wt_all, st).wait()
    plsc.subcore_barrier()

    def gfetch(k, b):
        pltpu.async_copy(infeat_hbm.at[src_all.at[k]], rows[b], sg[b])

    def process(k, b):
        pltpu.make_async_copy(
            infeat_hbm.at[src_all.at[k]], rows[b], sg[b]).wait()

        for g in range(CHUNK // L):
            w16 = wt_all[k, pl.ds(g * L, L)]
            for t in range(L):
                wb = jnp.full((L,), w16[t])
                e = g * L + t
                for j in range(D_OUT // L):
                    rows[b][e, pl.ds(j * L, L)] = (
                        rows[b][e, pl.ds(j * L, L)] * wb)

        pltpu.async_copy(rows[b], h_sh.at[dst_all.at[k]], sc[b], add=True)

    def scat_wait(b):
        pltpu.make_async_copy(rows[b], h_sh.at[dst_all.at[0]], sc[b]).wait()

    gfetch(0, 0)
    gfetch(1, 1)
    gfetch(2, 2)
    process(0, 0)
    gfetch(3, 3)

    def quad_body(j, carry):
        k4 = 4 * j + 1
        for t in range(4):
            k = k4 + t
            s = (1 + t) % 4
            process(k, s)

            @pl.when(k + 3 <= NCHUNK - 1)
            def _prefetch():
                nb = (s + 3) % 4
                scat_wait(nb)
                gfetch(k + 3, nb)
        return carry

    lax.fori_loop(0, (NCHUNK - 1) // 4, quad_body, 0)
    for s in range(4):
        scat_wait(s)

    plsc.subcore_barrier()

    @pl.when(sid < WB_TILES)
    def _writeback():
        pltpu.sync_copy(h_sh.at[pl.ds(r0, RPT)],
                        out_hbm.at[pl.ds(cid * N + r0, RPT)])


def kernel(feat, edge_index, edge_weight, node_graph_ids, W, b):
    del node_graph_ids
    b2 = b.reshape(1, D_OUT)
    in_feat, anchor_norm = _tc_prep(feat, W, b2)
    src2 = edge_index[0].reshape(E // CHUNK, CHUNK)
    dst2 = edge_index[1].reshape(E // CHUNK, CHUNK)
    wt2 = edge_weight.reshape(E // CHUNK, CHUNK)
    parts = _sc_edges(in_feat, src2, dst2, wt2)
    parts = parts.reshape(NC, N, D_OUT)
    h_norm, pooled_norm = _tc_final(parts, b2)
    return (h_norm, pooled_norm, anchor_norm)
